# baseline (speedup 1.0000x reference)
import jax
import jax.numpy as jnp
from jax import lax
from jax.experimental import pallas as pl
from jax.experimental.pallas import tpu as pltpu
from jax.experimental.pallas import tpu_sc as plsc

N_NODES = 10000
N_EDGES = 320000
NPAD = 10240
BM = 256
NSC = 2
NSUB = 16
NW = NSC * NSUB
BE = 56
EPW = 10080
EPAD = NW * EPW
NB = EPW // BE
NB2 = NB // 2
RPS = NPAD // NSUB
ZR = 32
WC = 128

_f32 = jnp.float32


def _sc_mesh():
    return plsc.VectorSubcoreMesh(core_axis_name="c", subcore_axis_name="s")


def _fill(buf, rows, cols, value):
    def frow(r, _):
        for j in range(cols // 16):
            buf[r, pl.ds(j * 16, 16)] = jnp.full((16,), value, _f32)
        return 0
    lax.fori_loop(0, rows, frow, 0)


def _cnt_pass(dst3):
    @pl.kernel(
        out_type=jax.ShapeDtypeStruct((NSC, NPAD, 128), _f32),
        mesh=_sc_mesh(),
        scratch_types=[
            pltpu.VMEM((NB, BE), jnp.int32),
            pltpu.VMEM((BE, 128), _f32),
            pltpu.VMEM((ZR, 128), _f32),
            pltpu.VMEM_SHARED((NPAD, 128), _f32),
        ],
    )
    def k(dst_hbm, out_hbm, didx, ones, zbuf, csh):
        cid = lax.axis_index("c")
        sid = lax.axis_index("s")
        wid = cid * NSUB + sid
        pltpu.sync_copy(dst_hbm.at[wid], didx)
        _fill(zbuf, ZR, 128, 0.0)
        _fill(ones, BE, 128, 1.0)

        qrow0 = pl.multiple_of(sid * RPS, ZR)
        for z in range(RPS // ZR):
            pltpu.sync_copy(zbuf, csh.at[pl.ds(qrow0 + z * ZR, ZR)])
        plsc.subcore_barrier()

        def bb(b, _):
            pltpu.sync_copy(ones, csh.at[didx.at[b]], add=True)
            return 0
        lax.fori_loop(0, NB, bb, 0)
        plsc.subcore_barrier()
        pltpu.sync_copy(csh.at[pl.ds(qrow0, RPS)],
                        out_hbm.at[cid, pl.ds(qrow0, RPS)])

    return k(dst3)


def _edge_pass(Gf, cidx, dst3, C):
    def body(gf_hbm, cidx_hbm, dst_hbm, out_hbm, cring, dring, zbuf,
             M0, R0, M1, R1, qsh, semI0, semI1, semG0, semS0, semG1, semS1):
        cid = lax.axis_index("c")
        sid = lax.axis_index("s")
        wid = cid * NSUB + sid
        _fill(zbuf, ZR, WC, 0.0)
        qrow0 = pl.multiple_of(sid * RPS, ZR)
        slot = ((M0, R0, semI0, semG0, semS0),
                (M1, R1, semI1, semG1, semS1))

        for c in range(C):
            def iload(b, semI):
                r = lax.rem(b, 8)
                pltpu.async_copy(cidx_hbm.at[c, wid, b], cring.at[r], semI)
                pltpu.async_copy(dst_hbm.at[wid, b], dring.at[r], semI)

            def iwait(b, semI):
                r = lax.rem(b, 8)
                pltpu.make_async_copy(cidx_hbm.at[c, wid, b], cring.at[r],
                                      semI).wait()
                pltpu.make_async_copy(dst_hbm.at[wid, b], dring.at[r],
                                      semI).wait()

            def gather(b, M, semG):
                r = lax.rem(b, 8)
                pltpu.async_copy(gf_hbm.at[cring.at[r]], M, semG)

            def gwait(b, M, semG):
                r = lax.rem(b, 8)
                pltpu.make_async_copy(gf_hbm.at[cring.at[r]], M, semG).wait()

            def compute(M, R):
                def rrow(r, _):
                    for j in range(WC // 16):
                        s_ = pl.ds(j * 16, 16)
                        R[r, s_] = jnp.maximum(M[r, s_] + M[r + BE, s_], 0.0)
                    return 0
                lax.fori_loop(0, BE, rrow, 0)

            def scat(b, R, semS):
                r = lax.rem(b, 8)
                pltpu.async_copy(R, qsh.at[dring.at[r]], semS, add=True)

            def swait(b, R, semS):
                r = lax.rem(b, 8)
                pltpu.make_async_copy(R, qsh.at[dring.at[r]], semS).wait()

            for b in range(4):
                iload(b, slot[b % 2][2])
            for b in range(2):
                M, _R, semI, semG, _semS = slot[b % 2]
                iwait(b, semI)
                gather(b, M, semG)

            for z in range(RPS // ZR):
                pltpu.sync_copy(zbuf, qsh.at[pl.ds(qrow0 + z * ZR, ZR)])
            plsc.subcore_barrier()

            def step(b, i, sl):
                M, R, semI, semG, semS = slot[sl]
                gwait(b, M, semG)

                @pl.when(i > 0)
                def _():
                    swait(b - 2, R, semS)

                @pl.when(b + 4 < NB)
                def _():
                    iload(b + 4, semI)
                compute(M, R)

                @pl.when(b + 2 < NB)
                def _():
                    iwait(b + 2, semI)
                    gather(b + 2, M, semG)
                scat(b, R, semS)

            def outer(i, _):
                step(i * 2, i, 0)
                step(i * 2 + 1, i, 1)
                return 0
            lax.fori_loop(0, NB2, outer, 0)
            swait(NB - 2, R0, semS0)
            swait(NB - 1, R1, semS1)
            plsc.subcore_barrier()
            pltpu.sync_copy(qsh.at[pl.ds(qrow0, RPS)],
                            out_hbm.at[c, cid, pl.ds(qrow0, RPS)])
            plsc.subcore_barrier()

    k = pl.kernel(
        body,
        out_type=jax.ShapeDtypeStruct((C, NSC, NPAD, WC), _f32),
        mesh=_sc_mesh(),
        scratch_types=[
            pltpu.VMEM((8, 2 * BE), jnp.int32),
            pltpu.VMEM((8, BE), jnp.int32),
            pltpu.VMEM((ZR, WC), _f32),
            pltpu.VMEM((2 * BE, WC), _f32),
            pltpu.VMEM((BE, WC), _f32),
            pltpu.VMEM((2 * BE, WC), _f32),
            pltpu.VMEM((BE, WC), _f32),
            pltpu.VMEM_SHARED((NPAD, WC), _f32),
            pltpu.SemaphoreType.DMA,
            pltpu.SemaphoreType.DMA,
            pltpu.SemaphoreType.DMA,
            pltpu.SemaphoreType.DMA,
            pltpu.SemaphoreType.DMA,
            pltpu.SemaphoreType.DMA,
        ],
    )
    return k(Gf, cidx, dst3)


def _d1(h, We1, be1, fi, fk, relu_in):
    C = fk // WC

    def body(h_ref, w_ref, b_ref, out_ref):
        hv = h_ref[...]
        if relu_in:
            hv = jnp.maximum(hv, 0.0)
        wtop = w_ref[:fi, :]
        wbot = w_ref[fi:, :]
        hA = jnp.dot(hv, wtop - wbot,
                     preferred_element_type=_f32) + b_ref[...]
        hB = jnp.dot(hv, wbot, preferred_element_type=_f32)
        for c in range(C):
            out_ref[c] = hA[:, c * WC:(c + 1) * WC]
            out_ref[C + c] = hB[:, c * WC:(c + 1) * WC]

    return pl.pallas_call(
        body,
        grid=(NPAD // BM,),
        in_specs=[
            pl.BlockSpec((BM, fi), lambda i: (i, 0)),
            pl.BlockSpec((2 * fi, fk), lambda i: (0, 0)),
            pl.BlockSpec((1, fk), lambda i: (0, 0)),
        ],
        out_specs=pl.BlockSpec((2 * C, BM, WC), lambda i: (0, i, 0)),
        out_shape=jax.ShapeDtypeStruct((2 * C, NPAD, WC), _f32),
    )(h, We1, be1)


def _d2(Qp, cntp, We2, be2, Wm1, bm1, Wm2, bm2, fk, nxt=None):
    C = fk // WC
    fh = fk // 2

    def body(*refs):
        (q_ref, c_ref, w2_ref, b2_ref, wm1_ref, bb1_ref, wm2_ref,
         bb2_ref) = refs[:8]
        if nxt is not None:
            w1_ref, b1_ref = refs[8:10]
            out_ref, g_ref = refs[10:]
        else:
            out_ref = refs[8]
        cnt = c_ref[0] + c_ref[1]
        inv = 1.0 / jnp.maximum(cnt, 1.0)
        msk = jnp.minimum(cnt, 1.0)[:, 0:1]
        qs = [(q_ref[c, 0] + q_ref[c, 1]) * inv[:, :WC] for c in range(C)]
        Qn = jnp.concatenate(qs, axis=1)
        agg = (jnp.dot(Qn, w2_ref[...], preferred_element_type=_f32)
               + b2_ref[...] * msk)
        a = jnp.maximum(agg, 0.0)
        t = jnp.maximum(
            jnp.dot(a, wm1_ref[...], preferred_element_type=_f32) + bb1_ref[...],
            0.0)
        out = (jnp.dot(t, wm2_ref[...], preferred_element_type=_f32)
               + bb2_ref[...])
        out_ref[...] = out
        if nxt is not None:
            fkn = nxt[2]
            Cn = fkn // WC
            hv = jnp.maximum(out, 0.0)
            wtop = w1_ref[:fk, :]
            wbot = w1_ref[fk:, :]
            hA = jnp.dot(hv, wtop - wbot, preferred_element_type=_f32) + b1_ref[...]
            hB = jnp.dot(hv, wbot, preferred_element_type=_f32)
            for c in range(Cn):
                g_ref[c] = hA[:, c * WC:(c + 1) * WC]
                g_ref[Cn + c] = hB[:, c * WC:(c + 1) * WC]

    in_specs = [
        pl.BlockSpec((C, NSC, BM, WC), lambda i: (0, 0, i, 0)),
        pl.BlockSpec((NSC, BM, 128), lambda i: (0, i, 0)),
        pl.BlockSpec((fk, fk), lambda i: (0, 0)),
        pl.BlockSpec((1, fk), lambda i: (0, 0)),
        pl.BlockSpec((fk, fh), lambda i: (0, 0)),
        pl.BlockSpec((1, fh), lambda i: (0, 0)),
        pl.BlockSpec((fh, fk), lambda i: (0, 0)),
        pl.BlockSpec((1, fk), lambda i: (0, 0)),
    ]
    args = [Qp, cntp, We2, be2, Wm1, bm1, Wm2, bm2]
    out_specs = pl.BlockSpec((BM, fk), lambda i: (i, 0))
    out_shape = jax.ShapeDtypeStruct((NPAD, fk), _f32)
    if nxt is not None:
        We1n, be1n, fkn = nxt
        Cn = fkn // WC
        in_specs += [
            pl.BlockSpec((2 * fk, fkn), lambda i: (0, 0)),
            pl.BlockSpec((1, fkn), lambda i: (0, 0)),
        ]
        args += [We1n, be1n.reshape(1, fkn)]
        out_specs = [out_specs,
                     pl.BlockSpec((2 * Cn, BM, WC), lambda i: (0, i, 0))]
        out_shape = [out_shape,
                     jax.ShapeDtypeStruct((2 * Cn, NPAD, WC), _f32)]
    return pl.pallas_call(
        body,
        grid=(NPAD // BM,),
        in_specs=in_specs,
        out_specs=out_specs,
        out_shape=out_shape,
    )(*args)


def _d3(outs, wslices, bo1, Wo2, bo2, fks):
    fo = Wo2.shape[1]

    def body(o1, o2, o3, o4, w1, w2, w3, w4, b1_ref, w5_ref, b2_ref, out_ref):
        acc = b1_ref[...]
        for o_ref, w_ref in ((o1, w1), (o2, w2), (o3, w3), (o4, w4)):
            acc = acc + jnp.dot(o_ref[...], w_ref[...],
                                preferred_element_type=_f32)
        y = jnp.maximum(acc, 0.0)
        out_ref[...] = (jnp.dot(y, w5_ref[...], preferred_element_type=_f32)
                        + b2_ref[...])

    in_specs = [pl.BlockSpec((BM, fk), lambda i: (i, 0)) for fk in fks]
    in_specs += [pl.BlockSpec((fk, fo), lambda i: (0, 0)) for fk in fks]
    in_specs += [
        pl.BlockSpec((1, fo), lambda i: (0, 0)),
        pl.BlockSpec((fo, fo), lambda i: (0, 0)),
        pl.BlockSpec((1, fo), lambda i: (0, 0)),
    ]
    return pl.pallas_call(
        body,
        grid=(NPAD // BM,),
        in_specs=in_specs,
        out_specs=pl.BlockSpec((BM, fo), lambda i: (i, 0)),
        out_shape=jax.ShapeDtypeStruct((NPAD, fo), _f32),
    )(*outs, *wslices, bo1, Wo2, bo2)


def kernel(x, edge_index, params):
    src = edge_index[0]
    dst = edge_index[1]
    npad_e = EPAD - N_EDGES
    fill = N_NODES + (jnp.arange(npad_e, dtype=jnp.int32) % (NPAD - N_NODES))
    src3 = jnp.concatenate([src, fill]).reshape(NW, NB, BE)
    dst3 = jnp.concatenate([dst, fill]).reshape(NW, NB, BE)
    h = jnp.pad(x, ((0, NPAD - N_NODES), (0, 0)))

    cntp = _cnt_pass(dst3)

    fis = (128, 256, 256, 256)
    fks = (256, 256, 256, 512)
    blocks = params["blocks"]
    outs = []
    G = _d1(h, blocks[0][0], blocks[0][1].reshape(1, fks[0]), fis[0], fks[0],
            relu_in=False)
    for bi, (We1, be1, We2, be2, Wm1, bm1, Wm2, bm2) in enumerate(blocks):
        fk = fks[bi]
        C = fk // WC
        Gf = G.reshape(2 * C * NPAD, WC)
        cidx = jnp.stack(
            [jnp.concatenate([dst3 + c * NPAD, src3 + (C + c) * NPAD],
                             axis=-1) for c in range(C)])
        Qp = _edge_pass(Gf, cidx, dst3, C)
        if bi + 1 < len(blocks):
            nxt = (blocks[bi + 1][0], blocks[bi + 1][1], fks[bi + 1])
        else:
            nxt = None
        res = _d2(Qp, cntp, We2, be2.reshape(1, fk), Wm1,
                  bm1.reshape(1, fk // 2), Wm2, bm2.reshape(1, fk), fk,
                  nxt=nxt)
        if nxt is not None:
            out_b, G = res
        else:
            out_b = res
        outs.append(out_b)

    Wo1, bo1, Wo2, bo2 = params["out"]
    splits = []
    r0 = 0
    for fk in fks:
        splits.append(Wo1[r0:r0 + fk])
        r0 += fk
    y = _d3(outs, splits, bo1.reshape(1, -1), Wo2, bo2.reshape(1, -1), fks)
    return y[:N_NODES]


# speedup vs baseline: 1.0062x; 1.0062x over previous
import jax
import jax.numpy as jnp
from jax import lax
from jax.experimental import pallas as pl
from jax.experimental.pallas import tpu as pltpu
from jax.experimental.pallas import tpu_sc as plsc

N_NODES = 10000
N_EDGES = 320000
NPAD = 10240
BM = 256
NSC = 2
NSUB = 16
NW = NSC * NSUB
BE = 56
EPW = 10080
EPAD = NW * EPW
NB = EPW // BE
NB2 = NB // 2
RPS = NPAD // NSUB
ZR = 32
WC = 128

_f32 = jnp.float32


def _sc_mesh():
    return plsc.VectorSubcoreMesh(core_axis_name="c", subcore_axis_name="s")


def _fill(buf, rows, cols, value):
    def frow(r, _):
        for j in range(cols // 16):
            buf[r, pl.ds(j * 16, 16)] = jnp.full((16,), value, _f32)
        return 0
    lax.fori_loop(0, rows, frow, 0)


def _cnt_pass(dst3):
    @pl.kernel(
        out_type=jax.ShapeDtypeStruct((NSC, NPAD, 128), _f32),
        mesh=_sc_mesh(),
        scratch_types=[
            pltpu.VMEM((NB, BE), jnp.int32),
            pltpu.VMEM((BE, 128), _f32),
            pltpu.VMEM((ZR, 128), _f32),
            pltpu.VMEM_SHARED((NPAD, 128), _f32),
        ],
    )
    def k(dst_hbm, out_hbm, didx, ones, zbuf, csh):
        cid = lax.axis_index("c")
        sid = lax.axis_index("s")
        wid = cid * NSUB + sid
        pltpu.sync_copy(dst_hbm.at[wid], didx)
        _fill(zbuf, ZR, 128, 0.0)
        _fill(ones, BE, 128, 1.0)

        qrow0 = pl.multiple_of(sid * RPS, ZR)
        for z in range(RPS // ZR):
            pltpu.sync_copy(zbuf, csh.at[pl.ds(qrow0 + z * ZR, ZR)])
        plsc.subcore_barrier()

        def bb(b, _):
            pltpu.sync_copy(ones, csh.at[didx.at[b]], add=True)
            return 0
        lax.fori_loop(0, NB, bb, 0)
        plsc.subcore_barrier()
        pltpu.sync_copy(csh.at[pl.ds(qrow0, RPS)],
                        out_hbm.at[cid, pl.ds(qrow0, RPS)])

    return k(dst3)


def _edge_pass(G, src3, dst3, C):
    def body(*refs):
        g_hbm = refs[0]
        src_hbm = refs[1]
        dst_hbm = refs[2]
        out_hbm = refs[3]
        (sidx8, didx8, zbuf, bufA0, bufB0, bufR0, bufA1, bufB1, bufR1, qsh,
         semI0, semI1, semA0, semB0, semS0, semA1, semB1, semS1) = refs[4:]
        hA = [g_hbm.at[c] for c in range(C)]
        hB = [g_hbm.at[C + c] for c in range(C)]

        cid = lax.axis_index("c")
        sid = lax.axis_index("s")
        wid = cid * NSUB + sid
        _fill(zbuf, ZR, WC, 0.0)
        qrow0 = pl.multiple_of(sid * RPS, ZR)
        slot = ((bufA0, bufB0, bufR0, semI0, semA0, semB0, semS0),
                (bufA1, bufB1, bufR1, semI1, semA1, semB1, semS1))

        def iload(b, semI):
            r = lax.rem(b, 8)
            pltpu.async_copy(src_hbm.at[wid, b], sidx8.at[r], semI)
            pltpu.async_copy(dst_hbm.at[wid, b], didx8.at[r], semI)

        def iwait(b, semI):
            r = lax.rem(b, 8)
            pltpu.make_async_copy(src_hbm.at[wid, b], sidx8.at[r],
                                  semI).wait()
            pltpu.make_async_copy(dst_hbm.at[wid, b], didx8.at[r],
                                  semI).wait()

        for c in range(C):
            def gather(b, bufA, bufB, semA, semB):
                r = lax.rem(b, 8)
                pltpu.async_copy(hA[c].at[didx8.at[r]], bufA, semA)
                pltpu.async_copy(hB[c].at[sidx8.at[r]], bufB, semB)

            def gwait(b, bufA, bufB, semA, semB):
                r = lax.rem(b, 8)
                pltpu.make_async_copy(hA[c].at[didx8.at[r]], bufA,
                                      semA).wait()
                pltpu.make_async_copy(hB[c].at[sidx8.at[r]], bufB,
                                      semB).wait()

            def compute(bufA, bufB, bufR):
                def rrow(r, _):
                    for j in range(WC // 16):
                        s_ = pl.ds(j * 16, 16)
                        bufR[r, s_] = jnp.maximum(bufA[r, s_] + bufB[r, s_],
                                                  0.0)
                    return 0
                lax.fori_loop(0, BE, rrow, 0)

            def scat(b, bufR, semS):
                r = lax.rem(b, 8)
                pltpu.async_copy(bufR, qsh.at[didx8.at[r]], semS, add=True)

            def swait(b, bufR, semS):
                r = lax.rem(b, 8)
                pltpu.make_async_copy(bufR, qsh.at[didx8.at[r]], semS).wait()

            for b in range(4):
                iload(b, slot[b % 2][3])
            for b in range(2):
                bufA, bufB, _bufR, semI, semA, semB, _semS = slot[b % 2]
                iwait(b, semI)
                gather(b, bufA, bufB, semA, semB)

            for z in range(RPS // ZR):
                pltpu.sync_copy(zbuf, qsh.at[pl.ds(qrow0 + z * ZR, ZR)])
            plsc.subcore_barrier()

            def step(b, i, sl):
                bufA, bufB, bufR, semI, semA, semB, semS = slot[sl]
                gwait(b, bufA, bufB, semA, semB)

                @pl.when(i > 0)
                def _():
                    swait(b - 2, bufR, semS)

                @pl.when(b + 4 < NB)
                def _():
                    iload(b + 4, semI)
                compute(bufA, bufB, bufR)

                @pl.when(b + 2 < NB)
                def _():
                    iwait(b + 2, semI)
                    gather(b + 2, bufA, bufB, semA, semB)
                scat(b, bufR, semS)

            def outer(i, _):
                step(i * 2, i, 0)
                step(i * 2 + 1, i, 1)
                return 0
            lax.fori_loop(0, NB2, outer, 0)
            swait(NB - 2, bufR0, semS0)
            swait(NB - 1, bufR1, semS1)
            plsc.subcore_barrier()
            pltpu.sync_copy(qsh.at[pl.ds(qrow0, RPS)],
                            out_hbm.at[c, cid, pl.ds(qrow0, RPS)])
            plsc.subcore_barrier()

    k = pl.kernel(
        body,
        out_type=jax.ShapeDtypeStruct((C, NSC, NPAD, WC), _f32),
        mesh=_sc_mesh(),
        scratch_types=[
            pltpu.VMEM((8, BE), jnp.int32),
            pltpu.VMEM((8, BE), jnp.int32),
            pltpu.VMEM((ZR, WC), _f32),
            pltpu.VMEM((BE, WC), _f32),
            pltpu.VMEM((BE, WC), _f32),
            pltpu.VMEM((BE, WC), _f32),
            pltpu.VMEM((BE, WC), _f32),
            pltpu.VMEM((BE, WC), _f32),
            pltpu.VMEM((BE, WC), _f32),
            pltpu.VMEM_SHARED((NPAD, WC), _f32),
            pltpu.SemaphoreType.DMA,
            pltpu.SemaphoreType.DMA,
            pltpu.SemaphoreType.DMA,
            pltpu.SemaphoreType.DMA,
            pltpu.SemaphoreType.DMA,
            pltpu.SemaphoreType.DMA,
            pltpu.SemaphoreType.DMA,
            pltpu.SemaphoreType.DMA,
        ],
    )
    return k(G, src3, dst3)


def _d1(h, We1, be1, fi, fk, relu_in):
    C = fk // WC

    def body(h_ref, w_ref, b_ref, out_ref):
        hv = h_ref[...]
        if relu_in:
            hv = jnp.maximum(hv, 0.0)
        wtop = w_ref[:fi, :]
        wbot = w_ref[fi:, :]
        hA = jnp.dot(hv, wtop - wbot,
                     preferred_element_type=_f32) + b_ref[...]
        hB = jnp.dot(hv, wbot, preferred_element_type=_f32)
        for c in range(C):
            out_ref[c] = hA[:, c * WC:(c + 1) * WC]
            out_ref[C + c] = hB[:, c * WC:(c + 1) * WC]

    return pl.pallas_call(
        body,
        grid=(NPAD // BM,),
        in_specs=[
            pl.BlockSpec((BM, fi), lambda i: (i, 0)),
            pl.BlockSpec((2 * fi, fk), lambda i: (0, 0)),
            pl.BlockSpec((1, fk), lambda i: (0, 0)),
        ],
        out_specs=pl.BlockSpec((2 * C, BM, WC), lambda i: (0, i, 0)),
        out_shape=jax.ShapeDtypeStruct((2 * C, NPAD, WC), _f32),
    )(h, We1, be1)


def _d2(Qp, cntp, We2, be2, Wm1, bm1, Wm2, bm2, fk, nxt=None):
    C = fk // WC
    fh = fk // 2

    def body(*refs):
        (q_ref, c_ref, w2_ref, b2_ref, wm1_ref, bb1_ref, wm2_ref,
         bb2_ref) = refs[:8]
        if nxt is not None:
            w1_ref, b1_ref = refs[8:10]
            out_ref, g_ref = refs[10:]
        else:
            out_ref = refs[8]
        cnt = c_ref[0] + c_ref[1]
        inv = 1.0 / jnp.maximum(cnt, 1.0)
        msk = jnp.minimum(cnt, 1.0)[:, 0:1]
        qs = [(q_ref[c, 0] + q_ref[c, 1]) * inv[:, :WC] for c in range(C)]
        Qn = jnp.concatenate(qs, axis=1)
        agg = (jnp.dot(Qn, w2_ref[...], preferred_element_type=_f32)
               + b2_ref[...] * msk)
        a = jnp.maximum(agg, 0.0)
        t = jnp.maximum(
            jnp.dot(a, wm1_ref[...], preferred_element_type=_f32) + bb1_ref[...],
            0.0)
        out = (jnp.dot(t, wm2_ref[...], preferred_element_type=_f32)
               + bb2_ref[...])
        out_ref[...] = out
        if nxt is not None:
            fkn = nxt[2]
            Cn = fkn // WC
            hv = jnp.maximum(out, 0.0)
            wtop = w1_ref[:fk, :]
            wbot = w1_ref[fk:, :]
            hA = jnp.dot(hv, wtop - wbot, preferred_element_type=_f32) + b1_ref[...]
            hB = jnp.dot(hv, wbot, preferred_element_type=_f32)
            for c in range(Cn):
                g_ref[c] = hA[:, c * WC:(c + 1) * WC]
                g_ref[Cn + c] = hB[:, c * WC:(c + 1) * WC]

    in_specs = [
        pl.BlockSpec((C, NSC, BM, WC), lambda i: (0, 0, i, 0)),
        pl.BlockSpec((NSC, BM, 128), lambda i: (0, i, 0)),
        pl.BlockSpec((fk, fk), lambda i: (0, 0)),
        pl.BlockSpec((1, fk), lambda i: (0, 0)),
        pl.BlockSpec((fk, fh), lambda i: (0, 0)),
        pl.BlockSpec((1, fh), lambda i: (0, 0)),
        pl.BlockSpec((fh, fk), lambda i: (0, 0)),
        pl.BlockSpec((1, fk), lambda i: (0, 0)),
    ]
    args = [Qp, cntp, We2, be2, Wm1, bm1, Wm2, bm2]
    out_specs = pl.BlockSpec((BM, fk), lambda i: (i, 0))
    out_shape = jax.ShapeDtypeStruct((NPAD, fk), _f32)
    if nxt is not None:
        We1n, be1n, fkn = nxt
        Cn = fkn // WC
        in_specs += [
            pl.BlockSpec((2 * fk, fkn), lambda i: (0, 0)),
            pl.BlockSpec((1, fkn), lambda i: (0, 0)),
        ]
        args += [We1n, be1n.reshape(1, fkn)]
        out_specs = [out_specs,
                     pl.BlockSpec((2 * Cn, BM, WC), lambda i: (0, i, 0))]
        out_shape = [out_shape,
                     jax.ShapeDtypeStruct((2 * Cn, NPAD, WC), _f32)]
    return pl.pallas_call(
        body,
        grid=(NPAD // BM,),
        in_specs=in_specs,
        out_specs=out_specs,
        out_shape=out_shape,
    )(*args)


def _d3(outs, wslices, bo1, Wo2, bo2, fks):
    fo = Wo2.shape[1]

    def body(o1, o2, o3, o4, w1, w2, w3, w4, b1_ref, w5_ref, b2_ref, out_ref):
        acc = b1_ref[...]
        for o_ref, w_ref in ((o1, w1), (o2, w2), (o3, w3), (o4, w4)):
            acc = acc + jnp.dot(o_ref[...], w_ref[...],
                                preferred_element_type=_f32)
        y = jnp.maximum(acc, 0.0)
        out_ref[...] = (jnp.dot(y, w5_ref[...], preferred_element_type=_f32)
                        + b2_ref[...])

    in_specs = [pl.BlockSpec((BM, fk), lambda i: (i, 0)) for fk in fks]
    in_specs += [pl.BlockSpec((fk, fo), lambda i: (0, 0)) for fk in fks]
    in_specs += [
        pl.BlockSpec((1, fo), lambda i: (0, 0)),
        pl.BlockSpec((fo, fo), lambda i: (0, 0)),
        pl.BlockSpec((1, fo), lambda i: (0, 0)),
    ]
    return pl.pallas_call(
        body,
        grid=(NPAD // BM,),
        in_specs=in_specs,
        out_specs=pl.BlockSpec((BM, fo), lambda i: (i, 0)),
        out_shape=jax.ShapeDtypeStruct((NPAD, fo), _f32),
    )(*outs, *wslices, bo1, Wo2, bo2)


def kernel(x, edge_index, params):
    src = edge_index[0]
    dst = edge_index[1]
    npad_e = EPAD - N_EDGES
    fill = N_NODES + (jnp.arange(npad_e, dtype=jnp.int32) % (NPAD - N_NODES))
    src3 = jnp.concatenate([src, fill]).reshape(NW, NB, BE)
    dst3 = jnp.concatenate([dst, fill]).reshape(NW, NB, BE)
    h = jnp.pad(x, ((0, NPAD - N_NODES), (0, 0)))

    cntp = _cnt_pass(dst3)

    fis = (128, 256, 256, 256)
    fks = (256, 256, 256, 512)
    blocks = params["blocks"]
    outs = []
    G = _d1(h, blocks[0][0], blocks[0][1].reshape(1, fks[0]), fis[0], fks[0],
            relu_in=False)
    for bi, (We1, be1, We2, be2, Wm1, bm1, Wm2, bm2) in enumerate(blocks):
        fk = fks[bi]
        Qp = _edge_pass(G, src3, dst3, fk // WC)
        if bi + 1 < len(blocks):
            nxt = (blocks[bi + 1][0], blocks[bi + 1][1], fks[bi + 1])
        else:
            nxt = None
        res = _d2(Qp, cntp, We2, be2.reshape(1, fk), Wm1,
                  bm1.reshape(1, fk // 2), Wm2, bm2.reshape(1, fk), fk,
                  nxt=nxt)
        if nxt is not None:
            out_b, G = res
        else:
            out_b = res
        outs.append(out_b)

    Wo1, bo1, Wo2, bo2 = params["out"]
    splits = []
    r0 = 0
    for fk in fks:
        splits.append(Wo1[r0:r0 + fk])
        r0 += fk
    y = _d3(outs, splits, bo1.reshape(1, -1), Wo2, bo2.reshape(1, -1), fks)
    return y[:N_NODES]
